# baseline (speedup 1.0000x reference)
import jax
import jax.numpy as jnp
from jax.experimental import pallas as pl
from jax.experimental.pallas import tpu as pltpu

N, D, H = 10000, 128, 256
BM = 200
HALF_BLOCKS = (N // 2) // BM


def _probe_kernel(a1_ref, a2_ref, o1_ref, o2_ref):
    o1_ref[:] = a1_ref[:, :H]
    o2_ref[:] = a2_ref[:, :H]


def kernel(x, adj, W, b):
    grid = (HALF_BLOCKS,)
    o1, o2 = pl.pallas_call(
        _probe_kernel,
        grid=grid,
        in_specs=[
            pl.BlockSpec((BM, N), lambda m: (m, 0)),
            pl.BlockSpec((BM, N), lambda m: (HALF_BLOCKS + m, 0)),
        ],
        out_specs=[
            pl.BlockSpec((BM, H), lambda m: (m, 0)),
            pl.BlockSpec((BM, H), lambda m: (m, 0)),
        ],
        out_shape=[
            jax.ShapeDtypeStruct((N // 2, H), jnp.float32),
            jax.ShapeDtypeStruct((N // 2, H), jnp.float32),
        ],
        compiler_params=pltpu.CompilerParams(
            dimension_semantics=("parallel",),
        ),
    )(adj, adj)
    return jnp.concatenate([o1, o2], axis=0)


# speedup vs baseline: 1.0393x; 1.0393x over previous
import jax
import jax.numpy as jnp
from jax.experimental import pallas as pl
from jax.experimental.pallas import tpu as pltpu

N, D, H = 10000, 128, 256
BM = 400


def _gcn_kernel(adj_ref, x_ref, W1_ref, W2_ref, b_ref, out_ref):
    m = pl.program_id(0)
    x_self = x_ref[pl.ds(m * BM, BM), :]
    agg = jnp.dot(adj_ref[:], x_ref[:], preferred_element_type=jnp.float32)
    z = jnp.dot(x_self, W1_ref[:], preferred_element_type=jnp.float32)
    z += jnp.dot(agg, W2_ref[:], preferred_element_type=jnp.float32)
    z += b_ref[:]
    out_ref[:] = jnp.maximum(z, 0.0)


def kernel(x, adj, W, b):
    W1 = W[:D]
    W2 = W[D:]
    b2 = b.reshape(1, H)
    grid = (N // BM,)
    return pl.pallas_call(
        _gcn_kernel,
        grid=grid,
        in_specs=[
            pl.BlockSpec((BM, N), lambda m: (m, 0)),
            pl.BlockSpec((N, D), lambda m: (0, 0)),
            pl.BlockSpec((D, H), lambda m: (0, 0)),
            pl.BlockSpec((D, H), lambda m: (0, 0)),
            pl.BlockSpec((1, H), lambda m: (0, 0)),
        ],
        out_specs=pl.BlockSpec((BM, H), lambda m: (m, 0)),
        out_shape=jax.ShapeDtypeStruct((N, H), jnp.float32),
        compiler_params=pltpu.CompilerParams(
            dimension_semantics=("parallel",),
        ),
    )(adj, x, W1, W2, b2)
